# Initial kernel scaffold; baseline (speedup 1.0000x reference)
import jax
import jax.numpy as jnp
from jax.experimental import pallas as pl


def kernel(x, expert_weights, token_indices, w1, w2, w3):
    raise NotImplementedError("write your pallas kernel here")


# speedup vs baseline: 1197.2615x; 1197.2615x over previous
import functools

import jax
import jax.numpy as jnp
from jax.experimental import pallas as pl
from jax.experimental.pallas import tpu as pltpu


def _swiglu_body(x_ref, wt_ref, w1_ref, w2_ref, w3_ref, eo_ref, wo_ref):
    xb = x_ref[...].astype(jnp.bfloat16)
    g = jnp.dot(xb, w1_ref[...], preferred_element_type=jnp.float32)
    u = jnp.dot(xb, w2_ref[...], preferred_element_type=jnp.float32)
    h = (g * jax.lax.logistic(g) * u).astype(jnp.bfloat16)
    eo = jnp.dot(h, w3_ref[...], preferred_element_type=jnp.float32)
    eo_ref[...] = eo
    wo_ref[...] = eo * wt_ref[...]


def _expert_mlp(xg, wts, w1, w2, w3):
    N, D = xg.shape
    dff = w1.shape[1]
    Tt = 512
    eo, wo = pl.pallas_call(
        _swiglu_body,
        grid=(N // Tt,),
        in_specs=[
            pl.BlockSpec((Tt, D), lambda i: (i, 0)),
            pl.BlockSpec((Tt, 1), lambda i: (i, 0)),
            pl.BlockSpec((D, dff), lambda i: (0, 0)),
            pl.BlockSpec((D, dff), lambda i: (0, 0)),
            pl.BlockSpec((dff, D), lambda i: (0, 0)),
        ],
        out_specs=[
            pl.BlockSpec((Tt, D), lambda i: (i, 0)),
            pl.BlockSpec((Tt, D), lambda i: (i, 0)),
        ],
        out_shape=[jax.ShapeDtypeStruct((N, D), jnp.float32)] * 2,
    )(xg, wts, w1.astype(jnp.bfloat16), w2.astype(jnp.bfloat16),
      w3.astype(jnp.bfloat16))
    return eo, wo


def kernel(x, expert_weights, token_indices, w1, w2, w3):
    B, S, D = x.shape
    E = len(w1)
    CAP = token_indices.shape[2]

    x_flat = x.reshape(B * S, D)
    gidx = token_indices + (jnp.arange(B, dtype=jnp.int32)[:, None, None] * S)

    expert_outputs = []
    out_flat = jnp.zeros((B * S, D), jnp.float32)
    for e in range(E):
        ge = gidx[:, e, :].reshape(-1)
        xg = jnp.take(x_flat, ge, axis=0)
        wts = expert_weights[:, e, :].reshape(-1, 1)
        eo, wo = _expert_mlp(xg, wts, w1[e], w2[e], w3[e])
        expert_outputs.append(eo.reshape(B, CAP, D))
        out_flat = out_flat.at[ge].add(wo)
    return out_flat.reshape(B, S, D), tuple(expert_outputs)


# speedup vs baseline: 1342.3501x; 1.1212x over previous
import functools

import jax
import jax.numpy as jnp
from jax import lax
from jax.experimental import pallas as pl
from jax.experimental.pallas import tpu as pltpu
from jax.experimental.pallas import tpu_sc as plsc

_NC = 2
_NS = 16
_NW = _NC * _NS


def _sc_gather(x_flat, gidx):
    N = gidx.shape[0]
    R, D = x_flat.shape
    per_w = N // _NW
    C = 32
    nch = per_w // C
    mesh = plsc.VectorSubcoreMesh(core_axis_name="c", subcore_axis_name="s")

    @functools.partial(
        pl.kernel, mesh=mesh,
        out_type=jax.ShapeDtypeStruct((N, D), jnp.float32),
        scratch_types=[
            pltpu.VMEM((C,), jnp.int32),
            pltpu.VMEM((C, D), jnp.float32),
            pltpu.SemaphoreType.DMA,
        ],
    )
    def k(x_hbm, idx_hbm, out_hbm, idx_v, rows_v, sem):
        wid = lax.axis_index("s") * _NC + lax.axis_index("c")
        base = wid * per_w

        def body(i, carry):
            off = base + i * C
            pltpu.sync_copy(idx_hbm.at[pl.ds(off, C)], idx_v)
            pltpu.async_copy(x_hbm.at[idx_v], rows_v, sem).wait()
            pltpu.sync_copy(rows_v, out_hbm.at[pl.ds(off, C)])
            return carry

        lax.fori_loop(0, nch, body, 0)

    return k(x_flat, gidx)


def _swiglu_body(x_ref, wt_ref, w1_ref, w2_ref, w3_ref, eo_ref, wo_ref):
    xb = x_ref[...].astype(jnp.bfloat16)
    g = jnp.dot(xb, w1_ref[...], preferred_element_type=jnp.float32)
    u = jnp.dot(xb, w2_ref[...], preferred_element_type=jnp.float32)
    h = (g * jax.lax.logistic(g) * u).astype(jnp.bfloat16)
    eo = jnp.dot(h, w3_ref[...], preferred_element_type=jnp.float32)
    eo_ref[...] = eo
    wo_ref[...] = eo * wt_ref[...]


def _expert_mlp(xg_full, wts, w1, w2, w3, e, E, B, CAP):
    D = xg_full.shape[1]
    dff = w1.shape[1]
    Tt = 512
    T = CAP // Tt
    eo, wo = pl.pallas_call(
        _swiglu_body,
        grid=(B, T),
        in_specs=[
            pl.BlockSpec((Tt, D), lambda b, t: (b * E * T + e * T + t, 0)),
            pl.BlockSpec((Tt, 1), lambda b, t: (b * T + t, 0)),
            pl.BlockSpec((D, dff), lambda b, t: (0, 0)),
            pl.BlockSpec((D, dff), lambda b, t: (0, 0)),
            pl.BlockSpec((dff, D), lambda b, t: (0, 0)),
        ],
        out_specs=[
            pl.BlockSpec((Tt, D), lambda b, t: (b * T + t, 0)),
            pl.BlockSpec((Tt, D), lambda b, t: (b * T + t, 0)),
        ],
        out_shape=[jax.ShapeDtypeStruct((B * CAP, D), jnp.float32)] * 2,
    )(xg_full, wts, w1.astype(jnp.bfloat16), w2.astype(jnp.bfloat16),
      w3.astype(jnp.bfloat16))
    return eo, wo


def kernel(x, expert_weights, token_indices, w1, w2, w3):
    B, S, D = x.shape
    E = len(w1)
    CAP = token_indices.shape[2]

    x_flat = x.reshape(B * S, D)
    gidx = token_indices + (jnp.arange(B, dtype=jnp.int32)[:, None, None] * S)
    xg = _sc_gather(x_flat, gidx.reshape(-1))

    expert_outputs = []
    out_flat = jnp.zeros((B * S, D), jnp.float32)
    for e in range(E):
        ge = gidx[:, e, :].reshape(-1)
        wts = expert_weights[:, e, :].reshape(-1, 1)
        eo, wo = _expert_mlp(xg, wts, w1[e], w2[e], w3[e], e, E, B, CAP)
        expert_outputs.append(eo.reshape(B, CAP, D))
        out_flat = out_flat.at[ge].add(wo)
    return out_flat.reshape(B, S, D), tuple(expert_outputs)
